# Initial kernel scaffold; baseline (speedup 1.0000x reference)
import jax
import jax.numpy as jnp
from jax.experimental import pallas as pl


def kernel(x_real, x_imag, apod_real, apod_imag, gu_vals_real, gu_vals_imag, phi_real, phi_imag, rdcf, gu_rows, gu_cols):
    raise NotImplementedError("write your pallas kernel here")


# speedup vs baseline: 2.2240x; 2.2240x over previous
import functools

import jax
import jax.numpy as jnp
from jax import lax
from jax.experimental import pallas as pl
from jax.experimental.pallas import tpu as pltpu
from jax.experimental.pallas import tpu_sc as plsc

_BATCH = 2
_NSLC = 5
_NCOILS = 1
_K = 6
_GX = 512
_GY = 512
_M = 36000
_NNZ_PER_ROW = 48
_C = _BATCH * _NSLC * _NCOILS
_CP = 16
_NGRID = _K * _GX * _GY

_NC = 2
_NS = 16
_NW = _NC * _NS
_MPAD = 36864
_ROWS_PER_W = _MPAD // _NW
_RCHUNK = 16
_STEPS = _ROWS_PER_W // _RCHUNK
_ECHUNK = _RCHUNK * _NNZ_PER_ROW
_NNZ_PAD = _MPAD * _NNZ_PER_ROW
_PPR = 4

_MB = 1024


def _sc_spmm(table, rowidx, lanebase, vals_r, vals_i):
    mesh = plsc.VectorSubcoreMesh(core_axis_name="c", subcore_axis_name="s")

    @functools.partial(
        pl.kernel,
        mesh=mesh,
        compiler_params=pltpu.CompilerParams(needs_layout_passes=False),
        out_type=[
            jax.ShapeDtypeStruct((_MPAD, _CP), jnp.float32),
            jax.ShapeDtypeStruct((_MPAD, _CP), jnp.float32),
        ],
        scratch_types=[
            pltpu.VMEM((_ECHUNK,), jnp.int32),
            pltpu.VMEM((_ECHUNK,), jnp.int32),
            pltpu.VMEM((_ECHUNK,), jnp.float32),
            pltpu.VMEM((_ECHUNK,), jnp.float32),
            pltpu.VMEM((_ECHUNK, 128), jnp.float32),
            pltpu.VMEM((_RCHUNK, _CP), jnp.float32),
            pltpu.VMEM((_RCHUNK, _CP), jnp.float32),
            pltpu.SemaphoreType.DMA,
        ],
    )
    def k(tbl_hbm, ridx_hbm, lb_hbm, vr_hbm, vi_hbm, outr_hbm, outi_hbm,
          idx_v, lb_v, valr_v, vali_v, rows_v, accr_v, acci_v, sem):
        wid = lax.axis_index("s") * _NC + lax.axis_index("c")
        lanes = lax.iota(jnp.int32, 16)

        def step(t, carry):
            rbase = wid * _ROWS_PER_W + t * _RCHUNK
            ebase = rbase * _NNZ_PER_ROW
            pltpu.sync_copy(ridx_hbm.at[pl.ds(ebase, _ECHUNK)], idx_v)
            pltpu.sync_copy(lb_hbm.at[pl.ds(ebase, _ECHUNK)], lb_v)
            pltpu.sync_copy(vr_hbm.at[pl.ds(ebase, _ECHUNK)], valr_v)
            pltpu.sync_copy(vi_hbm.at[pl.ds(ebase, _ECHUNK)], vali_v)
            pltpu.async_copy(tbl_hbm.at[idx_v], rows_v, sem).wait()

            def row(r, carry2):
                accr = jnp.zeros((16,), jnp.float32)
                acci = jnp.zeros((16,), jnp.float32)
                for j in range(_NNZ_PER_ROW):
                    e = r * _NNZ_PER_ROW + j
                    es = jnp.full((16,), e, jnp.int32)
                    vr = plsc.load_gather(valr_v, [es])
                    vi = plsc.load_gather(vali_v, [es])
                    lb = plsc.load_gather(lb_v, [es]) + lanes
                    ar = plsc.load_gather(rows_v, [es, lb])
                    ai = plsc.load_gather(rows_v, [es, lb + 16])
                    accr = accr + vr * ar - vi * ai
                    acci = acci + vr * ai + vi * ar
                rs = jnp.full((16,), r, jnp.int32)
                plsc.store_scatter(accr_v, [rs, lanes], accr)
                plsc.store_scatter(acci_v, [rs, lanes], acci)
                return carry2

            lax.fori_loop(0, _RCHUNK, row, 0)
            pltpu.sync_copy(accr_v, outr_hbm.at[pl.ds(rbase, _RCHUNK)])
            pltpu.sync_copy(acci_v, outi_hbm.at[pl.ds(rbase, _RCHUNK)])
            return carry

        lax.fori_loop(0, _STEPS, step, 0)

    return k(table, rowidx, lanebase, vals_r, vals_i)


def _mix_body(gr_ref, gi_ref, pr_ref, pi_ref, w_ref, yr_ref, yi_ref):
    w = w_ref[0, :]
    for b in range(_BATCH):
        pr_acc = jnp.zeros_like(w)
        pi_acc = jnp.zeros_like(w)
        grs = []
        gis = []
        for s in range(_NSLC):
            gr = gr_ref[b * _NSLC + s, :]
            gi = gi_ref[b * _NSLC + s, :]
            fr = pr_ref[s, :]
            fi = pi_ref[s, :]
            pr_acc = pr_acc + gr * fr - gi * fi
            pi_acc = pi_acc + gr * fi + gi * fr
            grs.append(gr)
            gis.append(gi)
        for r in range(_NSLC):
            fr = pr_ref[r, :]
            fi = pi_ref[r, :]
            mag = 1.0 - (fr * fr + fi * fi)
            yr = grs[r] * mag + pr_acc * fr + pi_acc * fi
            yi = gis[r] * mag + pi_acc * fr - pr_acc * fi
            yr_ref[b * _NSLC + r, :] = yr * w
            yi_ref[b * _NSLC + r, :] = yi * w


def _mix(gr_t, gi_t, phi_r, phi_i, w):
    grid = _MPAD // _MB
    spec_g = pl.BlockSpec((_C, _MB), lambda i: (0, i))
    spec_p = pl.BlockSpec((_NSLC, _MB), lambda i: (0, i))
    spec_w = pl.BlockSpec((1, _MB), lambda i: (0, i))
    return pl.pallas_call(
        _mix_body,
        grid=(grid,),
        in_specs=[spec_g, spec_g, spec_p, spec_p, spec_w],
        out_specs=[spec_g, spec_g],
        out_shape=[
            jax.ShapeDtypeStruct((_C, _MPAD), jnp.float32),
            jax.ShapeDtypeStruct((_C, _MPAD), jnp.float32),
        ],
    )(gr_t, gi_t, phi_r, phi_i, w)


def kernel(x_real, x_imag, apod_real, apod_imag, gu_vals_real, gu_vals_imag,
           phi_real, phi_imag, rdcf, gu_rows, gu_cols):
    x = lax.complex(x_real, x_imag)
    apod = lax.complex(apod_real, apod_imag)[None, None, None, None, :, :]
    f = jnp.fft.fft2(apod * x, s=(_GX, _GY), axes=(-2, -1)).astype(jnp.complex64)
    fl = f.reshape(_C, _NGRID)
    zpad = jnp.zeros((_CP - _C, _NGRID), jnp.float32)
    packed = jnp.concatenate(
        [jnp.real(fl), zpad, jnp.imag(fl), zpad], axis=0)
    table = packed.T.reshape(_NGRID // _PPR, _PPR * 32)

    pad_e = _NNZ_PAD - _M * _NNZ_PER_ROW
    cols_p = jnp.pad(gu_cols, (0, pad_e))
    rowidx = cols_p // _PPR
    lanebase = (cols_p % _PPR) * 32
    vr_p = jnp.pad(gu_vals_real, (0, pad_e))
    vi_p = jnp.pad(gu_vals_imag, (0, pad_e))
    g_r, g_i = _sc_spmm(table, rowidx, lanebase, vr_p, vi_p)

    pad_m = _MPAD - _M
    gr_t = g_r[:, :_C].T
    gi_t = g_i[:, :_C].T
    phi_r = jnp.pad(phi_real, ((0, 0), (0, pad_m)))
    phi_i = jnp.pad(phi_imag, ((0, 0), (0, pad_m)))
    w = jnp.pad(rdcf, (0, pad_m))[None, :]

    yr, yi = _mix(gr_t, gi_t, phi_r, phi_i, w)
    y = lax.complex(yr[:, :_M], yi[:, :_M])
    return y.reshape(_BATCH, _NSLC, _M, _NCOILS)
